# Initial kernel scaffold; baseline (speedup 1.0000x reference)
import jax
import jax.numpy as jnp
from jax.experimental import pallas as pl


def kernel(x, emb, W, b):
    raise NotImplementedError("write your pallas kernel here")


# speedup vs baseline: 39.0220x; 39.0220x over previous
import functools

import jax
import jax.numpy as jnp
from jax import lax
from jax.experimental import pallas as pl
from jax.experimental.pallas import tpu as pltpu
from jax.experimental.pallas import tpu_sc as plsc

NC = 2
NS = 16
NW = NC * NS
LANES = 16
CP = 16


def _table_matmul_body(a_ref, b_ref, o_ref):
    o_ref[...] = jnp.dot(a_ref[...], b_ref[...],
                         preferred_element_type=jnp.float32)


def _logsoftmax_body(a_ref, b_ref, o_ref):
    a = a_ref[...] + b_ref[...]
    m = jnp.max(a, axis=-1, keepdims=True)
    e = jnp.exp(a - m)
    s = jnp.sum(e, axis=-1, keepdims=True)
    o_ref[...] = a - m - jnp.log(s)


def _make_sc_lookup(B, L, V):
    bpw = B // NW
    groups = bpw // LANES
    tab_words = V * L * CP
    mesh = plsc.VectorSubcoreMesh(core_axis_name="c", subcore_axis_name="s",
                                  num_cores=NC, num_subcores=NS)

    @functools.partial(
        pl.kernel,
        out_type=jax.ShapeDtypeStruct((B * CP,), jnp.float32),
        mesh=mesh,
        scratch_types=[
            pltpu.VMEM((tab_words,), jnp.float32),
            pltpu.VMEM((bpw * L,), jnp.int32),
            pltpu.VMEM((bpw * CP,), jnp.float32),
        ],
        compiler_params=pltpu.CompilerParams(needs_layout_passes=False),
    )
    def sc_lookup(table_hbm, x_hbm, out_hbm, table_v, x_v, out_v):
        wid = lax.axis_index("s") * NC + lax.axis_index("c")
        pltpu.sync_copy(table_hbm, table_v)
        pltpu.sync_copy(x_hbm.at[pl.ds(wid * (bpw * L), bpw * L)], x_v)

        iota = lax.iota(jnp.int32, LANES)
        zeros = jnp.zeros((LANES,), jnp.float32)
        for g in range(groups):
            row_ids = g * LANES + iota
            x_base = row_ids * L
            o_base = row_ids * CP

            def body(l, accs):
                xv = plsc.load_gather(x_v, [x_base + l])
                addr = xv * (L * CP) + l * CP
                return tuple(
                    accs[c] + plsc.load_gather(table_v, [addr + c])
                    for c in range(len(accs))
                )

            accs = lax.fori_loop(
                0, L, body,
                tuple(zeros for _ in range(10)))
            for c in range(10):
                plsc.store_scatter(out_v, [o_base + c], accs[c])
            for c in range(10, CP):
                plsc.store_scatter(out_v, [o_base + c], zeros)
        pltpu.sync_copy(out_v, out_hbm.at[pl.ds(wid * (bpw * CP), bpw * CP)])

    return sc_lookup


def kernel(x, emb, W, b):
    B, L = x.shape
    V, D = emb.shape
    C = W.shape[1]

    W_rp = jnp.pad(W.reshape(L, D, C).transpose(1, 0, 2),
                   ((0, 0), (0, 0), (0, CP - C))).reshape(D, L * CP)
    emb_pad = jnp.pad(emb, ((0, 8 - V), (0, 0)))

    T2 = pl.pallas_call(
        _table_matmul_body,
        out_shape=jax.ShapeDtypeStruct((8, L * CP), jnp.float32),
    )(emb_pad, W_rp)
    table = T2[:V].reshape(V * L * CP)

    logits = _make_sc_lookup(B, L, V)(
        table, x.astype(jnp.int32).reshape(B * L)).reshape(B, CP)

    b_pad = jnp.concatenate(
        [b, jnp.full((CP - C,), -1e30, jnp.float32)]).reshape(1, CP)
    out = pl.pallas_call(
        _logsoftmax_body,
        out_shape=jax.ShapeDtypeStruct((B, CP), jnp.float32),
    )(logits, b_pad)
    return out[:, :C]
